# Initial kernel scaffold; baseline (speedup 1.0000x reference)
import jax
import jax.numpy as jnp
from jax.experimental import pallas as pl


def kernel(x, Wg, w1, b1, w2, b2):
    raise NotImplementedError("write your pallas kernel here")


# speedup vs baseline: 1.0382x; 1.0382x over previous
import jax
import jax.numpy as jnp
from jax import lax
from jax.experimental import pallas as pl
from jax.experimental.pallas import tpu as pltpu


def _moe_body(x_ref, wg_ref, w1_ref, b1_ref, w2_ref, b2_ref, out_ref,
              probs_scr, idx_ce, g_ce, idx_ec, g_ec):
    e = pl.program_id(0)
    n, d = x_ref.shape
    num_e = wg_ref.shape[1]
    cap = idx_ce.shape[0]

    @pl.when(e == 0)
    def _router():
        tokens = x_ref[...]
        logits = jnp.dot(tokens, wg_ref[...],
                         preferred_element_type=jnp.float32)
        m = jnp.max(logits, axis=1, keepdims=True)
        p = jnp.exp(logits - m)
        probs_scr[...] = p / jnp.sum(p, axis=1, keepdims=True)

        def topk_step(k, carry):
            pm = probs_scr[...]
            mx = jnp.max(pm, axis=0, keepdims=True)
            rows = lax.broadcasted_iota(jnp.int32, (n, num_e), 0)
            cand = jnp.where(pm == mx, rows, n)
            am = jnp.min(cand, axis=0, keepdims=True)
            g_ce[pl.ds(k, 1), :] = mx
            idx_ce[pl.ds(k, 1), :] = am
            probs_scr[...] = jnp.where(rows == am, -jnp.inf, pm)
            return carry

        lax.fori_loop(0, cap, topk_step, 0)
        idx_ec[...] = idx_ce[...].T
        g_ec[...] = g_ce[...].T

    idx_row = idx_ec[pl.ds(e, 1), :]
    g_row = g_ec[pl.ds(e, 1), :]
    rows_n = lax.broadcasted_iota(jnp.int32, (n, cap), 0)
    oh = (rows_n == idx_row).astype(jnp.float32)
    disp = lax.dot_general(oh, x_ref[...], (((0,), (0,)), ((), ())),
                           preferred_element_type=jnp.float32)
    h = jnp.dot(disp, w1_ref[0], preferred_element_type=jnp.float32)
    h = jax.nn.gelu(h + b1_ref[pl.ds(e, 1), :])
    oe = jnp.dot(h, w2_ref[0], preferred_element_type=jnp.float32)
    oe = oe + b2_ref[pl.ds(e, 1), :]
    contrib = jnp.dot(oh * g_row, oe,
                      preferred_element_type=jnp.float32)

    @pl.when(e == 0)
    def _init():
        out_ref[...] = contrib

    @pl.when(e != 0)
    def _acc():
        out_ref[...] = out_ref[...] + contrib


def _moe(tokens, Wg, w1, b1, w2, b2, *, interpret=False):
    n, d = tokens.shape
    num_e = Wg.shape[1]
    f = w1.shape[2]
    cap = 2 * n // num_e

    return pl.pallas_call(
        _moe_body,
        grid=(num_e,),
        in_specs=[
            pl.BlockSpec((n, d), lambda e: (0, 0)),
            pl.BlockSpec((d, num_e), lambda e: (0, 0)),
            pl.BlockSpec((1, d, f), lambda e: (e, 0, 0)),
            pl.BlockSpec((num_e, f), lambda e: (0, 0)),
            pl.BlockSpec((1, f, d), lambda e: (e, 0, 0)),
            pl.BlockSpec((num_e, d), lambda e: (0, 0)),
        ],
        out_specs=pl.BlockSpec((n, d), lambda e: (0, 0)),
        out_shape=jax.ShapeDtypeStruct((n, d), jnp.float32),
        scratch_shapes=[
            pltpu.VMEM((n, num_e), jnp.float32),
            pltpu.VMEM((cap, num_e), jnp.int32),
            pltpu.VMEM((cap, num_e), jnp.float32),
            pltpu.VMEM((num_e, cap), jnp.int32),
            pltpu.VMEM((num_e, cap), jnp.float32),
        ],
        compiler_params=pltpu.CompilerParams(
            dimension_semantics=("arbitrary",),
        ),
        interpret=interpret,
    )(tokens, Wg, w1, b1, w2, b2)


def kernel(x, Wg, w1, b1, w2, b2):
    bb, ss, dd = x.shape
    out = _moe(x.reshape(bb * ss, dd), Wg, w1, b1, w2, b2)
    return out.reshape(bb, ss, dd)
